# baseline (speedup 1.0000x reference)
import functools

import jax
import jax.numpy as jnp
from jax import lax
from jax.experimental import pallas as pl
from jax.experimental.pallas import tpu as pltpu
from jax.experimental.pallas import tpu_sc as plsc

N = 10000
E = 160000
O = 8
C = 64
K = 64
P = 8


BM = 1600
RC = 64


def _spatial_body(kb_ref, w2_ref, sp_ref):
    for j in range(BM // RC):
        v = kb_ref[pl.ds(j * RC * 4, RC * 4), :]
        sp_ref[pl.ds(j * RC * 4, RC * 4), :] = jnp.dot(
            v, w2_ref[...], preferred_element_type=jnp.float32)


NS = 16
NCORE = 2
EPT = E // NS
S = 2000
NSCAN = EPT // S
G = 32
RANGE = 1792
NPASS = 3
ACC_ROWS = RANGE * 4 + 64
FL = 64


def _sc_conv_body(x_hbm, sp_hbm, src_hbm, dst_hbm, out_hbm,
                  dstv, srcv, f_eid, f_src, f_ld,
                  eid4, src4, ld4, sp_rows, x_rows, fl_buf,
                  acc, sem1, sem2):
    cid = lax.axis_index("c")
    sid = lax.axis_index("s")
    iota = lax.iota(jnp.int32, 16)
    zero16 = jnp.zeros((16,), jnp.float32)

    for p in range(NPASS):
        lo = cid * (NPASS * RANGE) + p * RANGE
        hi = lo + RANGE
        lo4 = lo * 4

        for g in range(FL):
            for cc in range(8):
                fl_buf[g, pl.ds(cc * 16, 16)] = zero16
        for k in range(RANGE * 4 // NS // FL):
            pltpu.sync_copy(
                fl_buf, acc.at[pl.ds(sid * (RANGE * 4 // NS) + k * FL, FL)])

        plsc.subcore_barrier()

        for j in range(NSCAN):
            base = sid * EPT + j * S
            pltpu.sync_copy(dst_hbm.at[pl.ds(base, S)], dstv)
            pltpu.sync_copy(src_hbm.at[pl.ds(base, S)], srcv)

            @pl.loop(0, S // 16, init_carry=jnp.int32(0))
            def off(i, off):
                d16 = dstv[pl.ds(i * 16, 16)]
                s16 = srcv[pl.ds(i * 16, 16)]
                m = (d16 >= lo) & (d16 < hi)
                eid16 = base + i * 16 + iota
                plsc.store_compressed(f_eid.at[pl.ds(off, 16)], eid16, mask=m)
                plsc.store_compressed(f_src.at[pl.ds(off, 16)], s16, mask=m)
                plsc.store_compressed(f_ld.at[pl.ds(off, 16)], d16 - lo, mask=m)
                return off + plsc.all_reduce_population_count(m)[0]

            pad_e = sid * 16 + iota
            pad_ld = jnp.full((16,), RANGE, jnp.int32) + sid
            for t in range(G // 16):
                f_eid[pl.ds(off + t * 16, 16)] = pad_e
                f_src[pl.ds(off + t * 16, 16)] = pad_e
                f_ld[pl.ds(off + t * 16, 16)] = pad_ld

            @pl.loop(0, (off + G - 1) // G)
            def _(k):
                kb = k * G
                for t in range(G // 16):
                    e16 = f_eid[pl.ds(kb + t * 16, 16)]
                    s16 = f_src[pl.ds(kb + t * 16, 16)]
                    l16 = f_ld[pl.ds(kb + t * 16, 16)]
                    for rr in range(4):
                        pos = t * 64 + iota * 4 + rr
                        plsc.store_scatter(eid4, [pos], e16 * 4 + rr)
                        plsc.store_scatter(src4, [pos], s16 * 4 + rr)
                        plsc.store_scatter(ld4, [pos], l16 * 4 + rr)
                cp1 = pltpu.async_copy(sp_hbm.at[eid4], sp_rows, sem1)
                cp2 = pltpu.async_copy(x_hbm.at[src4], x_rows, sem2)
                cp1.wait()
                cp2.wait()

                @pl.loop(0, 4 * G)
                def _(g):
                    for cc in range(8):
                        sl = pl.ds(cc * 16, 16)
                        sp_rows[g, sl] = sp_rows[g, sl] * x_rows[g, sl]

                pltpu.sync_copy(sp_rows, acc.at[ld4], add=True)

        plsc.subcore_barrier()

        nval = jnp.clip((N * 4 - (lo4 + sid * (RANGE * 4 // NS))) // FL,
                        0, RANGE * 4 // NS // FL)

        @pl.loop(0, nval)
        def _(k):
            r0 = sid * (RANGE * 4 // NS) + k * FL
            pltpu.sync_copy(acc.at[pl.ds(r0, FL)], fl_buf)
            pltpu.sync_copy(fl_buf, out_hbm.at[pl.ds(lo4 + r0, FL)])


NC_BLK = 200


def _fiber_body(x1_ref, fkb_ref, wf_ref, bias_ref, out_ref):
    f = jnp.dot(fkb_ref[...], wf_ref[...], preferred_element_type=jnp.float32)
    for p in range(P):
        acc = x1_ref[0] * f[p * O]
        for o in range(1, O):
            acc = acc + x1_ref[o] * f[p * O + o]
        out_ref[p] = acc * (1.0 / O) + bias_ref[0]


def kernel(x, kernel_basis, fiber_kernel_basis, edge_index, W_kernel, W_fiber, bias):
    zero = jnp.zeros((K, C), jnp.float32)
    w2 = jnp.block([[W_kernel, zero], [zero, W_kernel]])

    kbq = jnp.concatenate(
        [kernel_basis[:, 0:4, :], kernel_basis[:, 4:8, :]], axis=2
    ).reshape(E * 4, 128)
    sp = pl.pallas_call(
        _spatial_body,
        grid=(E // BM,),
        in_specs=[
            pl.BlockSpec((BM * 4, 128), lambda g: (g, 0)),
            pl.BlockSpec((128, 128), lambda g: (0, 0)),
        ],
        out_specs=pl.BlockSpec((BM * 4, 128), lambda g: (g, 0)),
        out_shape=jax.ShapeDtypeStruct((E * 4, 128), jnp.float32),
    )(kbq, w2)

    xp = jnp.concatenate([x[:, 0:4, :], x[:, 4:8, :]], axis=2).reshape(N * 4, 128)
    src = edge_index[0]
    dst = edge_index[1]

    mesh = plsc.VectorSubcoreMesh(
        core_axis_name="c", subcore_axis_name="s",
        num_cores=NCORE, num_subcores=NS)
    x1p = pl.kernel(
        _sc_conv_body,
        out_type=jax.ShapeDtypeStruct((N * 4, 128), jnp.float32),
        mesh=mesh,
        compiler_params=pltpu.CompilerParams(needs_layout_passes=False),
        scratch_types=[
            pltpu.VMEM((S,), jnp.int32),
            pltpu.VMEM((S,), jnp.int32),
            pltpu.VMEM((S + G,), jnp.int32),
            pltpu.VMEM((S + G,), jnp.int32),
            pltpu.VMEM((S + G,), jnp.int32),
            pltpu.VMEM((4 * G,), jnp.int32),
            pltpu.VMEM((4 * G,), jnp.int32),
            pltpu.VMEM((4 * G,), jnp.int32),
            pltpu.VMEM((4 * G, 128), jnp.float32),
            pltpu.VMEM((4 * G, 128), jnp.float32),
            pltpu.VMEM((FL, 128), jnp.float32),
            pltpu.VMEM_SHARED((ACC_ROWS, 128), jnp.float32),
            pltpu.SemaphoreType.DMA,
            pltpu.SemaphoreType.DMA,
        ],
    )(xp, sp, src, dst)

    fkb = fiber_kernel_basis.reshape(P * O, K)
    bias2 = bias.reshape(1, C)
    x1om = (x1p.reshape(N, 4, 2, C).transpose(2, 1, 0, 3)
            .reshape(O, N, C))
    x2om = pl.pallas_call(
        _fiber_body,
        grid=(N // NC_BLK,),
        in_specs=[
            pl.BlockSpec((O, NC_BLK, C), lambda g: (0, g, 0)),
            pl.BlockSpec((P * O, K), lambda g: (0, 0)),
            pl.BlockSpec((K, C), lambda g: (0, 0)),
            pl.BlockSpec((1, C), lambda g: (0, 0)),
        ],
        out_specs=pl.BlockSpec((P, NC_BLK, C), lambda g: (0, g, 0)),
        out_shape=jax.ShapeDtypeStruct((P, N, C), jnp.float32),
    )(x1om, fkb, W_fiber, bias2)

    return x2om.transpose(1, 0, 2)


# speedup vs baseline: 1.0532x; 1.0532x over previous
import functools

import jax
import jax.numpy as jnp
from jax import lax
from jax.experimental import pallas as pl
from jax.experimental.pallas import tpu as pltpu
from jax.experimental.pallas import tpu_sc as plsc

N = 10000
E = 160000
O = 8
C = 64
K = 64
P = 8


BM = 1600
RC = 64


def _spatial_body(kb_ref, w2_ref, sp_ref):
    for j in range(BM // RC):
        v = kb_ref[pl.ds(j * RC, RC)]
        c = jnp.concatenate([v[:, 0:4, :], v[:, 4:8, :]], axis=2)
        d = jnp.reshape(c, (RC * 4, 128))
        sp_ref[pl.ds(j * RC * 4, RC * 4), :] = jnp.dot(
            d, w2_ref[...], preferred_element_type=jnp.float32)


NS = 16
NCORE = 2
EPT = E // NS
S = 2000
NSCAN = EPT // S
G = 32
RANGE = 1792
NPASS = 3
ACC_ROWS = RANGE * 4 + 64
FL = 64


def _sc_conv_body(x_hbm, sp_hbm, src_hbm, dst_hbm, out_hbm,
                  dstv, srcv, f_eid, f_src, f_ld,
                  eid4, src4, ld4, sp_rows, x_rows, fl_buf,
                  acc, sem1, sem2):
    cid = lax.axis_index("c")
    sid = lax.axis_index("s")
    iota = lax.iota(jnp.int32, 16)
    zero16 = jnp.zeros((16,), jnp.float32)

    for p in range(NPASS):
        lo = cid * (NPASS * RANGE) + p * RANGE
        hi = lo + RANGE
        lo4 = lo * 4

        for g in range(FL):
            for cc in range(8):
                fl_buf[g, pl.ds(cc * 16, 16)] = zero16
        for k in range(RANGE * 4 // NS // FL):
            pltpu.sync_copy(
                fl_buf, acc.at[pl.ds(sid * (RANGE * 4 // NS) + k * FL, FL)])

        plsc.subcore_barrier()

        for j in range(NSCAN):
            base = sid * EPT + j * S
            pltpu.sync_copy(dst_hbm.at[pl.ds(base, S)], dstv)
            pltpu.sync_copy(src_hbm.at[pl.ds(base, S)], srcv)

            @pl.loop(0, S // 16, init_carry=jnp.int32(0))
            def off(i, off):
                d16 = dstv[pl.ds(i * 16, 16)]
                s16 = srcv[pl.ds(i * 16, 16)]
                m = (d16 >= lo) & (d16 < hi)
                eid16 = base + i * 16 + iota
                plsc.store_compressed(f_eid.at[pl.ds(off, 16)], eid16, mask=m)
                plsc.store_compressed(f_src.at[pl.ds(off, 16)], s16, mask=m)
                plsc.store_compressed(f_ld.at[pl.ds(off, 16)], d16 - lo, mask=m)
                return off + plsc.all_reduce_population_count(m)[0]

            pad_e = sid * 16 + iota
            pad_ld = jnp.full((16,), RANGE, jnp.int32) + sid
            for t in range(G // 16):
                f_eid[pl.ds(off + t * 16, 16)] = pad_e
                f_src[pl.ds(off + t * 16, 16)] = pad_e
                f_ld[pl.ds(off + t * 16, 16)] = pad_ld

            @pl.loop(0, (off + G - 1) // G)
            def _(k):
                kb = k * G
                for t in range(G // 16):
                    e16 = f_eid[pl.ds(kb + t * 16, 16)]
                    s16 = f_src[pl.ds(kb + t * 16, 16)]
                    l16 = f_ld[pl.ds(kb + t * 16, 16)]
                    for rr in range(4):
                        pos = t * 64 + iota * 4 + rr
                        plsc.store_scatter(eid4, [pos], e16 * 4 + rr)
                        plsc.store_scatter(src4, [pos], s16 * 4 + rr)
                        plsc.store_scatter(ld4, [pos], l16 * 4 + rr)
                cp1 = pltpu.async_copy(sp_hbm.at[eid4], sp_rows, sem1)
                cp2 = pltpu.async_copy(x_hbm.at[src4], x_rows, sem2)
                cp1.wait()
                cp2.wait()

                @pl.loop(0, 4 * G)
                def _(g):
                    for cc in range(8):
                        sl = pl.ds(cc * 16, 16)
                        sp_rows[g, sl] = sp_rows[g, sl] * x_rows[g, sl]

                pltpu.sync_copy(sp_rows, acc.at[ld4], add=True)

        plsc.subcore_barrier()

        nval = jnp.clip((N * 4 - (lo4 + sid * (RANGE * 4 // NS))) // FL,
                        0, RANGE * 4 // NS // FL)

        @pl.loop(0, nval)
        def _(k):
            r0 = sid * (RANGE * 4 // NS) + k * FL
            pltpu.sync_copy(acc.at[pl.ds(r0, FL)], fl_buf)
            pltpu.sync_copy(fl_buf, out_hbm.at[pl.ds(lo4 + r0, FL)])


NC_BLK = 200


def _fiber_body(x1_ref, fkb_ref, wf_ref, bias_ref, out_ref):
    f = jnp.dot(fkb_ref[...], wf_ref[...], preferred_element_type=jnp.float32)
    for p in range(P):
        acc = x1_ref[0] * f[p * O]
        for o in range(1, O):
            acc = acc + x1_ref[o] * f[p * O + o]
        out_ref[p] = acc * (1.0 / O) + bias_ref[0]


def kernel(x, kernel_basis, fiber_kernel_basis, edge_index, W_kernel, W_fiber, bias):
    zero = jnp.zeros((K, C), jnp.float32)
    w2 = jnp.block([[W_kernel, zero], [zero, W_kernel]])

    sp = pl.pallas_call(
        _spatial_body,
        grid=(E // BM,),
        in_specs=[
            pl.BlockSpec((BM, O, K), lambda g: (g, 0, 0)),
            pl.BlockSpec((128, 128), lambda g: (0, 0)),
        ],
        out_specs=pl.BlockSpec((BM * 4, 128), lambda g: (g, 0)),
        out_shape=jax.ShapeDtypeStruct((E * 4, 128), jnp.float32),
    )(kernel_basis, w2)

    xp = jnp.concatenate([x[:, 0:4, :], x[:, 4:8, :]], axis=2).reshape(N * 4, 128)
    src = edge_index[0]
    dst = edge_index[1]

    mesh = plsc.VectorSubcoreMesh(
        core_axis_name="c", subcore_axis_name="s",
        num_cores=NCORE, num_subcores=NS)
    x1p = pl.kernel(
        _sc_conv_body,
        out_type=jax.ShapeDtypeStruct((N * 4, 128), jnp.float32),
        mesh=mesh,
        compiler_params=pltpu.CompilerParams(needs_layout_passes=False),
        scratch_types=[
            pltpu.VMEM((S,), jnp.int32),
            pltpu.VMEM((S,), jnp.int32),
            pltpu.VMEM((S + G,), jnp.int32),
            pltpu.VMEM((S + G,), jnp.int32),
            pltpu.VMEM((S + G,), jnp.int32),
            pltpu.VMEM((4 * G,), jnp.int32),
            pltpu.VMEM((4 * G,), jnp.int32),
            pltpu.VMEM((4 * G,), jnp.int32),
            pltpu.VMEM((4 * G, 128), jnp.float32),
            pltpu.VMEM((4 * G, 128), jnp.float32),
            pltpu.VMEM((FL, 128), jnp.float32),
            pltpu.VMEM_SHARED((ACC_ROWS, 128), jnp.float32),
            pltpu.SemaphoreType.DMA,
            pltpu.SemaphoreType.DMA,
        ],
    )(xp, sp, src, dst)

    fkb = fiber_kernel_basis.reshape(P * O, K)
    bias2 = bias.reshape(1, C)
    x1om = (x1p.reshape(N, 4, 2, C).transpose(2, 1, 0, 3)
            .reshape(O, N, C))
    x2om = pl.pallas_call(
        _fiber_body,
        grid=(N // NC_BLK,),
        in_specs=[
            pl.BlockSpec((O, NC_BLK, C), lambda g: (0, g, 0)),
            pl.BlockSpec((P * O, K), lambda g: (0, 0)),
            pl.BlockSpec((K, C), lambda g: (0, 0)),
            pl.BlockSpec((1, C), lambda g: (0, 0)),
        ],
        out_specs=pl.BlockSpec((P, NC_BLK, C), lambda g: (0, g, 0)),
        out_shape=jax.ShapeDtypeStruct((P, N, C), jnp.float32),
    )(x1om, fkb, W_fiber, bias2)

    return x2om.transpose(1, 0, 2)
